# baseline (speedup 1.0000x reference)
import jax
import jax.numpy as jnp
import numpy as np
from jax.experimental import pallas as pl
from jax.experimental.pallas import tpu as pltpu

_N = 10000
_E = 160000
_R = 16
_D = 256
_H = 4
_HD = 64
_BN = 400
_NBLK = _N // _BN


def _dense_body(s_ref, cnt_ref, x_ref, w_ref, root_ref, bias_ref,
                attf_ref, e1_ref, e1t_ref, hout_ref, alpha_ref, acc_ref):
    r = pl.program_id(1)
    inv = 1.0 / jnp.maximum(cnt_ref[...], 1.0)
    lane = jax.lax.broadcasted_iota(jnp.int32, (_BN, _R), 1)
    invr = jnp.sum(jnp.where(lane == r, inv, 0.0), axis=1, keepdims=True)
    m = s_ref[...] * invr
    part = jnp.dot(m, w_ref[0], preferred_element_type=jnp.float32)

    @pl.when(r == 0)
    def _init():
        acc_ref[...] = part

    @pl.when(r > 0)
    def _accum():
        acc_ref[...] = acc_ref[...] + part

    @pl.when(r == _R - 1)
    def _epilogue():
        agg = acc_ref[...] + jnp.dot(x_ref[...], root_ref[...],
                                     preferred_element_type=jnp.float32)
        h = agg + bias_ref[...]
        p = h * attf_ref[...]
        score8 = jnp.dot(p, e1_ref[...], preferred_element_type=jnp.float32)
        score = score8[:, :_H]
        mx = jnp.max(score, axis=1, keepdims=True)
        ex = jnp.exp(score - mx)
        alpha = ex / jnp.sum(ex, axis=1, keepdims=True)
        e1t = e1t_ref[...][:_H]
        aexp = jnp.dot(alpha, e1t, preferred_element_type=jnp.float32)
        hout_ref[...] = h * aexp
        alpha_ref[...] = jnp.concatenate(
            [alpha, jnp.zeros((_BN, 8 - _H), jnp.float32)], axis=1)


def _dense_call(S2, cnt2, x, w_all, root, bias2, attf, e1, e1t):
    return pl.pallas_call(
        _dense_body,
        grid=(_NBLK, _R),
        in_specs=[
            pl.BlockSpec((_BN, _D), lambda i, r: (r * _NBLK + i, 0)),
            pl.BlockSpec((_BN, _R), lambda i, r: (i, 0)),
            pl.BlockSpec((_BN, _D), lambda i, r: (i, 0)),
            pl.BlockSpec((1, _D, _D), lambda i, r: (r, 0, 0)),
            pl.BlockSpec((_D, _D), lambda i, r: (0, 0)),
            pl.BlockSpec((1, _D), lambda i, r: (0, 0)),
            pl.BlockSpec((1, _D), lambda i, r: (0, 0)),
            pl.BlockSpec((_D, 8), lambda i, r: (0, 0)),
            pl.BlockSpec((8, _D), lambda i, r: (0, 0)),
        ],
        out_specs=[
            pl.BlockSpec((_BN, _D), lambda i, r: (i, 0)),
            pl.BlockSpec((_BN, 8), lambda i, r: (i, 0)),
        ],
        out_shape=[
            jax.ShapeDtypeStruct((_N, _D), jnp.float32),
            jax.ShapeDtypeStruct((_N, 8), jnp.float32),
        ],
        scratch_shapes=[pltpu.VMEM((_BN, _D), jnp.float32)],
    )(S2, cnt2, x, w_all, root, bias2, attf, e1, e1t)


def kernel(x, edge_index, edge_type, bases, comp, root, bias, att):
    src = edge_index[0]
    dst = edge_index[1]
    seg2 = edge_type * _N + dst
    S2 = jax.ops.segment_sum(x[src], seg2, num_segments=_R * _N)
    seg = dst * _R + edge_type
    cnt = jax.ops.segment_sum(jnp.ones((_E,), jnp.float32), seg,
                              num_segments=_N * _R)

    w_all = jnp.einsum('rb,bdo->rdo', comp, bases)
    bias2 = bias.reshape(1, _D)
    attf = att.reshape(1, _H * _HD)
    head = np.repeat(np.arange(_H), _HD)
    e1 = np.zeros((_D, 8), np.float32)
    e1[np.arange(_D), head] = 1.0
    e1t = np.zeros((8, _D), np.float32)
    e1t[head, np.arange(_D)] = 1.0
    hout, alpha8 = _dense_call(S2, cnt.reshape(_N, _R), x, w_all, root,
                               bias2, attf, jnp.asarray(e1), jnp.asarray(e1t))
    return hout, alpha8[:, :_H]


# speedup vs baseline: 1.0030x; 1.0030x over previous
import jax
import jax.numpy as jnp
import numpy as np
from jax.experimental import pallas as pl
from jax.experimental.pallas import tpu as pltpu

_N = 10000
_E = 160000
_R = 16
_D = 256
_H = 4
_HD = 64
_BN = 400
_NBLK = _N // _BN


def _dense_body(s_ref, cnt_ref, x_ref, w_ref, root_ref, bias_ref,
                attf_ref, e1_ref, e1t_ref, hout_ref, alpha_ref, acc_ref):
    r = pl.program_id(1)
    inv = 1.0 / jnp.maximum(cnt_ref[...], 1.0)
    lane = jax.lax.broadcasted_iota(jnp.int32, (_BN, _R), 1)
    invr = jnp.sum(jnp.where(lane == r, inv, 0.0), axis=1, keepdims=True)
    m = s_ref[...] * invr
    part = jnp.dot(m, w_ref[0], preferred_element_type=jnp.float32)

    @pl.when(r == 0)
    def _init():
        acc_ref[...] = part

    @pl.when(r > 0)
    def _accum():
        acc_ref[...] = acc_ref[...] + part

    @pl.when(r == _R - 1)
    def _epilogue():
        agg = acc_ref[...] + jnp.dot(x_ref[...], root_ref[...],
                                     preferred_element_type=jnp.float32)
        h = agg + bias_ref[...]
        p = h * attf_ref[...]
        score8 = jnp.dot(p, e1_ref[...], preferred_element_type=jnp.float32)
        score = score8[:, :_H]
        mx = jnp.max(score, axis=1, keepdims=True)
        ex = jnp.exp(score - mx)
        alpha = ex / jnp.sum(ex, axis=1, keepdims=True)
        e1t = e1t_ref[...][:_H]
        aexp = jnp.dot(alpha, e1t, preferred_element_type=jnp.float32)
        hout_ref[...] = h * aexp
        alpha_ref[...] = jnp.concatenate(
            [alpha, jnp.zeros((_BN, 8 - _H), jnp.float32)], axis=1)


def _dense_call(S2, cnt2, x, w_all, root, bias2, attf, e1, e1t):
    return pl.pallas_call(
        _dense_body,
        grid=(_NBLK, _R),
        in_specs=[
            pl.BlockSpec((_BN, _D), lambda i, r: (r * _NBLK + i, 0)),
            pl.BlockSpec((_BN, _R), lambda i, r: (i, 0)),
            pl.BlockSpec((_BN, _D), lambda i, r: (i, 0)),
            pl.BlockSpec((1, _D, _D), lambda i, r: (r, 0, 0)),
            pl.BlockSpec((_D, _D), lambda i, r: (0, 0)),
            pl.BlockSpec((1, _D), lambda i, r: (0, 0)),
            pl.BlockSpec((1, _D), lambda i, r: (0, 0)),
            pl.BlockSpec((_D, 8), lambda i, r: (0, 0)),
            pl.BlockSpec((8, _D), lambda i, r: (0, 0)),
        ],
        out_specs=[
            pl.BlockSpec((_BN, _D), lambda i, r: (i, 0)),
            pl.BlockSpec((_BN, 8), lambda i, r: (i, 0)),
        ],
        out_shape=[
            jax.ShapeDtypeStruct((_N, _D), jnp.float32),
            jax.ShapeDtypeStruct((_N, 8), jnp.float32),
        ],
        scratch_shapes=[pltpu.VMEM((_BN, _D), jnp.float32)],
    )(S2, cnt2, x, w_all, root, bias2, attf, e1, e1t)


def kernel(x, edge_index, edge_type, bases, comp, root, bias, att):
    src = edge_index[0]
    dst = edge_index[1]
    seg2 = edge_type * _N + dst
    S2 = jax.ops.segment_sum(x[src], seg2, num_segments=_R * _N)
    cnt = jax.ops.segment_sum(jnp.ones((_E,), jnp.float32), seg2,
                              num_segments=_R * _N).reshape(_R, _N).T

    w_all = jnp.einsum('rb,bdo->rdo', comp, bases)
    bias2 = bias.reshape(1, _D)
    attf = att.reshape(1, _H * _HD)
    head = np.repeat(np.arange(_H), _HD)
    e1 = np.zeros((_D, 8), np.float32)
    e1[np.arange(_D), head] = 1.0
    e1t = np.zeros((8, _D), np.float32)
    e1t[head, np.arange(_D)] = 1.0
    hout, alpha8 = _dense_call(S2, cnt, x, w_all, root,
                               bias2, attf, jnp.asarray(e1), jnp.asarray(e1t))
    return hout, alpha8[:, :_H]


# speedup vs baseline: 1.1341x; 1.1307x over previous
import jax
import jax.numpy as jnp
import numpy as np
from jax.experimental import pallas as pl
from jax.experimental.pallas import tpu as pltpu

_N = 10000
_E = 160000
_R = 16
_D = 256
_H = 4
_HD = 64
_BN = 400
_NBLK = _N // _BN


def _dense_body(s_ref, cnt_ref, x_ref, w_ref, root_ref, bias_ref,
                attf_ref, e1_ref, e1t_ref, hout_ref, alpha_ref):
    inv = 1.0 / jnp.maximum(cnt_ref[...], 1.0)
    agg = jnp.dot(x_ref[...], root_ref[...], preferred_element_type=jnp.float32)
    for r in range(_R):
        m = s_ref[r] * inv[:, r:r + 1]
        agg = agg + jnp.dot(m, w_ref[r], preferred_element_type=jnp.float32)
    h = agg + bias_ref[...]
    p = h * attf_ref[...]
    score8 = jnp.dot(p, e1_ref[...], preferred_element_type=jnp.float32)
    score = score8[:, :_H]
    mx = jnp.max(score, axis=1, keepdims=True)
    ex = jnp.exp(score - mx)
    alpha = ex / jnp.sum(ex, axis=1, keepdims=True)
    e1t = e1t_ref[...][:_H]
    aexp = jnp.dot(alpha, e1t, preferred_element_type=jnp.float32)
    hout_ref[...] = h * aexp
    alpha_ref[...] = jnp.concatenate(
        [alpha, jnp.zeros((_BN, 8 - _H), jnp.float32)], axis=1)


def _dense_call(S3, cnt2, x, w_all, root, bias2, attf, e1, e1t):
    return pl.pallas_call(
        _dense_body,
        grid=(_NBLK,),
        in_specs=[
            pl.BlockSpec((_R, _BN, _D), lambda i: (0, i, 0)),
            pl.BlockSpec((_BN, _R), lambda i: (i, 0)),
            pl.BlockSpec((_BN, _D), lambda i: (i, 0)),
            pl.BlockSpec((_R, _D, _D), lambda i: (0, 0, 0)),
            pl.BlockSpec((_D, _D), lambda i: (0, 0)),
            pl.BlockSpec((1, _D), lambda i: (0, 0)),
            pl.BlockSpec((1, _D), lambda i: (0, 0)),
            pl.BlockSpec((_D, 8), lambda i: (0, 0)),
            pl.BlockSpec((8, _D), lambda i: (0, 0)),
        ],
        out_specs=[
            pl.BlockSpec((_BN, _D), lambda i: (i, 0)),
            pl.BlockSpec((_BN, 8), lambda i: (i, 0)),
        ],
        out_shape=[
            jax.ShapeDtypeStruct((_N, _D), jnp.float32),
            jax.ShapeDtypeStruct((_N, 8), jnp.float32),
        ],
    )(S3, cnt2, x, w_all, root, bias2, attf, e1, e1t)


def kernel(x, edge_index, edge_type, bases, comp, root, bias, att):
    src = edge_index[0]
    dst = edge_index[1]
    seg2 = edge_type * _N + dst
    S2 = jax.ops.segment_sum(x[src], seg2, num_segments=_R * _N)
    cnt = jax.ops.segment_sum(jnp.ones((_E,), jnp.float32), seg2,
                              num_segments=_R * _N).reshape(_R, _N).T

    w_all = jnp.einsum('rb,bdo->rdo', comp, bases)
    bias2 = bias.reshape(1, _D)
    attf = att.reshape(1, _H * _HD)
    head = np.repeat(np.arange(_H), _HD)
    e1 = np.zeros((_D, 8), np.float32)
    e1[np.arange(_D), head] = 1.0
    e1t = np.zeros((8, _D), np.float32)
    e1t[head, np.arange(_D)] = 1.0
    hout, alpha8 = _dense_call(S2.reshape(_R, _N, _D), cnt, x, w_all, root,
                               bias2, attf, jnp.asarray(e1), jnp.asarray(e1t))
    return hout, alpha8[:, :_H]


# speedup vs baseline: 1.1448x; 1.0094x over previous
import jax
import jax.numpy as jnp
import numpy as np
from jax.experimental import pallas as pl

_N = 10000
_E = 160000
_R = 16
_D = 256
_H = 4
_HD = 64
_BN = 1000
_NBLK = _N // _BN


def _dense_body(s_ref, cnt_ref, x_ref, w_ref, root_ref, bias_ref,
                attf_ref, e1_ref, e1t_ref, hout_ref, alpha_ref):
    inv = 1.0 / jnp.maximum(cnt_ref[...], 1.0)
    agg = jnp.dot(x_ref[...], root_ref[...], preferred_element_type=jnp.float32)
    for r in range(_R):
        m = s_ref[r] * inv[:, r:r + 1]
        agg = agg + jnp.dot(m, w_ref[r], preferred_element_type=jnp.float32)
    h = agg + bias_ref[...]
    p = h * attf_ref[...]
    score8 = jnp.dot(p, e1_ref[...], preferred_element_type=jnp.float32)
    score = score8[:, :_H]
    mx = jnp.max(score, axis=1, keepdims=True)
    ex = jnp.exp(score - mx)
    alpha = ex / jnp.sum(ex, axis=1, keepdims=True)
    e1t = e1t_ref[...][:_H]
    aexp = jnp.dot(alpha, e1t, preferred_element_type=jnp.float32)
    hout_ref[...] = h * aexp
    alpha_ref[...] = jnp.concatenate(
        [alpha, jnp.zeros((_BN, 8 - _H), jnp.float32)], axis=1)


def _dense_call(S3, cnt2, x, w_all, root, bias2, attf, e1, e1t):
    return pl.pallas_call(
        _dense_body,
        grid=(_NBLK,),
        in_specs=[
            pl.BlockSpec((_R, _BN, _D), lambda i: (0, i, 0)),
            pl.BlockSpec((_BN, _R), lambda i: (i, 0)),
            pl.BlockSpec((_BN, _D), lambda i: (i, 0)),
            pl.BlockSpec((_R, _D, _D), lambda i: (0, 0, 0)),
            pl.BlockSpec((_D, _D), lambda i: (0, 0)),
            pl.BlockSpec((1, _D), lambda i: (0, 0)),
            pl.BlockSpec((1, _D), lambda i: (0, 0)),
            pl.BlockSpec((_D, 8), lambda i: (0, 0)),
            pl.BlockSpec((8, _D), lambda i: (0, 0)),
        ],
        out_specs=[
            pl.BlockSpec((_BN, _D), lambda i: (i, 0)),
            pl.BlockSpec((_BN, 8), lambda i: (i, 0)),
        ],
        out_shape=[
            jax.ShapeDtypeStruct((_N, _D), jnp.float32),
            jax.ShapeDtypeStruct((_N, 8), jnp.float32),
        ],
    )(S3, cnt2, x, w_all, root, bias2, attf, e1, e1t)


def kernel(x, edge_index, edge_type, bases, comp, root, bias, att):
    src = edge_index[0]
    dst = edge_index[1]
    seg2 = edge_type * _N + dst
    S2 = jax.ops.segment_sum(x[src], seg2, num_segments=_R * _N)
    cnt = jax.ops.segment_sum(jnp.ones((_E,), jnp.float32), seg2,
                              num_segments=_R * _N).reshape(_R, _N).T

    w_all = jnp.einsum('rb,bdo->rdo', comp, bases)
    bias2 = bias.reshape(1, _D)
    attf = att.reshape(1, _H * _HD)
    head = np.repeat(np.arange(_H), _HD)
    e1 = np.zeros((_D, 8), np.float32)
    e1[np.arange(_D), head] = 1.0
    e1t = np.zeros((8, _D), np.float32)
    e1t[head, np.arange(_D)] = 1.0
    hout, alpha8 = _dense_call(S2.reshape(_R, _N, _D), cnt, x, w_all, root,
                               bias2, attf, jnp.asarray(e1), jnp.asarray(e1t))
    return hout, alpha8[:, :_H]
